# baseline (speedup 1.0000x reference)
import functools

import jax
import jax.numpy as jnp
from jax import lax
from jax.experimental import pallas as pl
from jax.experimental.pallas import tpu as pltpu
from jax.experimental.pallas import tpu_sc as plsc

N = 10000
E = 320000
F_IN = 128
OUT = 128

NC = 2
NS = 16
L = 16
FW = 8

K = 80
EDGES_PER_TILE = E // (NC * NS)
CHUNKS = EDGES_PER_TILE // K
TRIPLES = (CHUNKS - 2) // 3
CBE = 2000
CBC = CBE // K
NPAD = 10240
ROWS_PT = NPAD // NS
NRB = ROWS_PT // K

NBLK = 1000
GRID = N // NBLK


def _dense_body(x_ref, w_ref, a_ref, b_ref, seq_ref, f_ref):
    s = jnp.dot(x_ref[...], w_ref[...], preferred_element_type=jnp.float32)
    seq_ref[...] = s
    f_ref[...] = jnp.dot(s, a_ref[...],
                         preferred_element_type=jnp.float32) + b_ref[...]


def _combine_body(acc_ref, den_ref, b_ref, o_ref):
    num = acc_ref[0] + acc_ref[1]
    den = den_ref[0, :, 0:1] + den_ref[1, :, 0:1]
    v = num / (den + 1e-16) + b_ref[...]
    o_ref[...] = jnp.where(v > 0, v, jnp.exp(jnp.minimum(v, 0.0)) - 1.0)


def _sc_body(seq_hbm, f1_hbm, f2_hbm, ei_hbm, acc_out, den_out,
             row_ib, col_ib, rows0, rows1, rows2,
             ri0, ri1, ri2, ci0, ci1, ci2, fg0, fg1, fg2,
             f2g0, f2g1, f2g2, exr0, exr1, exr2,
             acc_sh, den_sh, g0, g1, g2, s0, s1, s2):
    cid = lax.axis_index("c")
    sid = lax.axis_index("s")
    rows = (rows0, rows1, rows2)
    ris = (ri0, ri1, ri2)
    cis = (ci0, ci1, ci2)
    fgs = (fg0, fg1, fg2)
    f2gs = (f2g0, f2g1, f2g2)
    exrs = (exr0, exr1, exr2)
    gsems = (g0, g1, g2)
    ssems = (s0, s1, s2)

    zeros16 = jnp.zeros((L,), jnp.float32)
    zero16i = jnp.zeros((L,), jnp.int32)
    one16i = jnp.full((L,), 1, jnp.int32)
    lane = lax.iota(jnp.int32, L)

    def zero_body(i, _):
        for c in range(OUT // L):
            rows0[i, pl.ds(c * L, L)] = zeros16
        return 0
    lax.fori_loop(0, K, zero_body, 0)
    zr16 = lax.shift_right_logical(lane, 3)
    zc16 = lax.bitwise_and(lane, 7)
    for b in range(3):
        def zero_exr(i, _):
            plsc.store_scatter(exrs[b], [zr16 + 2 * i, zc16], zeros16)
            return 0
        lax.fori_loop(0, K // 2, zero_exr, 0)
    for b in range(NRB):
        r0 = sid * ROWS_PT + b * K
        pltpu.sync_copy(rows0, acc_sh.at[pl.ds(r0, K)])
        pltpu.sync_copy(exr0, den_sh.at[pl.ds(r0, K)])
    plsc.subcore_barrier()

    base_e = (cid * NS + sid) * EDGES_PER_TILE

    def issue_gathers(c, b):
        @pl.when(lax.rem(c, CBC) == 0)
        def _():
            off = base_e + (c // CBC) * CBE
            pltpu.sync_copy(ei_hbm.at[0, pl.ds(off, CBE)], row_ib)
            pltpu.sync_copy(ei_hbm.at[1, pl.ds(off, CBE)], col_ib)
        base = lax.rem(c, CBC) * K
        for i in range(K // L):
            ris[b][pl.ds(i * L, L)] = row_ib[pl.ds(base + i * L, L)]
            cis[b][pl.ds(i * L, L)] = col_ib[pl.ds(base + i * L, L)]
        pltpu.async_copy(seq_hbm.at[cis[b]], rows[b], gsems[b])
        pltpu.async_copy(f1_hbm.at[ris[b]], fgs[b], gsems[b])
        pltpu.async_copy(f2_hbm.at[cis[b]], f2gs[b], gsems[b])

    def wait_gathers(b):
        pltpu.make_async_copy(seq_hbm.at[cis[b]], rows[b], gsems[b]).wait()
        pltpu.make_async_copy(f1_hbm.at[ris[b]], fgs[b], gsems[b]).wait()
        pltpu.make_async_copy(f2_hbm.at[cis[b]], f2gs[b], gsems[b]).wait()

    def process(b):
        rows_v = rows[b]
        fg_v = fgs[b]
        exr_v = exrs[b]
        for i in range(K // L):
            e16 = lane + (i * L)
            f1g = fg_v[pl.ds(i * L, L)]
            f2g = f2gs[b][pl.ds(i * L, L)]
            lg = f1g + f2g
            lr = jnp.where(lg > 0, lg, 0.2 * lg)
            plsc.store_scatter(exr_v, [e16, zero16i], jnp.exp(lr))

        @plsc.parallel_loop(0, K, 1, unroll=4)
        def _scale(e):
            exb = plsc.load_gather(exr_v, [jnp.full((L,), e, jnp.int32),
                                           zero16i])
            for g in range(OUT // L):
                rows_v[e, pl.ds(g * L, L)] = rows_v[e, pl.ds(g * L, L)] * exb

    def issue_scatters(b):
        pltpu.async_copy(rows[b], acc_sh.at[ris[b]], ssems[b], add=True)
        pltpu.async_copy(exrs[b], den_sh.at[ris[b]], ssems[b], add=True)

    def wait_scatters(b):
        pltpu.make_async_copy(rows[b], acc_sh.at[ris[b]], ssems[b]).wait()
        pltpu.make_async_copy(exrs[b], den_sh.at[ris[b]], ssems[b]).wait()

    for b in range(3):
        issue_gathers(b, b)

    def triple_body(t, _):
        c0 = 3 * t
        for b in range(3):
            wait_gathers(b)
            process(b)
            issue_scatters(b)
        for b in range(3):
            wait_scatters(b)
        for b in range(3):
            cn = c0 + 3 + b
            @pl.when(cn < CHUNKS)
            def _():
                issue_gathers(cn, b)
        return 0

    lax.fori_loop(0, TRIPLES, triple_body, 0)

    for b in range(CHUNKS - 3 * TRIPLES):
        wait_gathers(b)
        process(b)
        issue_scatters(b)
    for b in range(CHUNKS - 3 * TRIPLES):
        wait_scatters(b)

    plsc.subcore_barrier()

    for b in range(NRB):
        r0 = sid * ROWS_PT + b * K
        pltpu.sync_copy(acc_sh.at[pl.ds(r0, K)], rows0)
        pltpu.sync_copy(rows0, acc_out.at[cid, pl.ds(r0, K)])
        pltpu.sync_copy(den_sh.at[pl.ds(r0, K)], exr0)
        pltpu.sync_copy(exr0, den_out.at[cid, pl.ds(r0, K)])


_sc_edge_kernel = functools.partial(
    pl.kernel,
    out_type=(jax.ShapeDtypeStruct((NC, NPAD, OUT), jnp.float32),
              jax.ShapeDtypeStruct((NC, NPAD, FW), jnp.float32)),
    mesh=plsc.VectorSubcoreMesh(core_axis_name="c", subcore_axis_name="s"),
    compiler_params=pltpu.CompilerParams(
        use_tc_tiling_on_sc=False, needs_layout_passes=False),
    scratch_types=[
        pltpu.VMEM((CBE,), jnp.int32),
        pltpu.VMEM((CBE,), jnp.int32),
        pltpu.VMEM((K, OUT), jnp.float32),
        pltpu.VMEM((K, OUT), jnp.float32),
        pltpu.VMEM((K, OUT), jnp.float32),
        pltpu.VMEM((K,), jnp.int32),
        pltpu.VMEM((K,), jnp.int32),
        pltpu.VMEM((K,), jnp.int32),
        pltpu.VMEM((K,), jnp.int32),
        pltpu.VMEM((K,), jnp.int32),
        pltpu.VMEM((K,), jnp.int32),
        pltpu.VMEM((K,), jnp.float32),
        pltpu.VMEM((K,), jnp.float32),
        pltpu.VMEM((K,), jnp.float32),
        pltpu.VMEM((K,), jnp.float32),
        pltpu.VMEM((K,), jnp.float32),
        pltpu.VMEM((K,), jnp.float32),
        pltpu.VMEM((K, FW), jnp.float32),
        pltpu.VMEM((K, FW), jnp.float32),
        pltpu.VMEM((K, FW), jnp.float32),
        pltpu.VMEM_SHARED((NPAD, OUT), jnp.float32),
        pltpu.VMEM_SHARED((NPAD, FW), jnp.float32),
        pltpu.SemaphoreType.DMA,
        pltpu.SemaphoreType.DMA,
        pltpu.SemaphoreType.DMA,
        pltpu.SemaphoreType.DMA,
        pltpu.SemaphoreType.DMA,
        pltpu.SemaphoreType.DMA,
    ],
)(_sc_body)


def kernel(x, edge_index, W, a1, b1, a2, b2, bias_out):
    xs = jnp.squeeze(x, 0)
    A = jnp.zeros((OUT, FW), jnp.float32).at[:, 0].set(a1[:, 0]).at[:, 1].set(a2[:, 0])
    bvec = jnp.zeros((1, FW), jnp.float32).at[0, 0].set(b1[0] + b2[0])

    seq, ft = pl.pallas_call(
        _dense_body,
        grid=(GRID,),
        in_specs=[
            pl.BlockSpec((NBLK, F_IN), lambda i: (i, 0)),
            pl.BlockSpec((F_IN, OUT), lambda i: (0, 0)),
            pl.BlockSpec((OUT, FW), lambda i: (0, 0)),
            pl.BlockSpec((1, FW), lambda i: (0, 0)),
        ],
        out_specs=[
            pl.BlockSpec((NBLK, OUT), lambda i: (i, 0)),
            pl.BlockSpec((NBLK, FW), lambda i: (i, 0)),
        ],
        out_shape=[
            jax.ShapeDtypeStruct((N, OUT), jnp.float32),
            jax.ShapeDtypeStruct((N, FW), jnp.float32),
        ],
    )(xs, W, A, bvec)

    f1 = ft[:, 0]
    f2 = ft[:, 1]
    acc, den = _sc_edge_kernel(seq, f1, f2, edge_index)

    out = pl.pallas_call(
        _combine_body,
        grid=(GRID,),
        in_specs=[
            pl.BlockSpec((NC, NBLK, OUT), lambda i: (0, i, 0)),
            pl.BlockSpec((NC, NBLK, FW), lambda i: (0, i, 0)),
            pl.BlockSpec((1, OUT), lambda i: (0, 0)),
        ],
        out_specs=pl.BlockSpec((NBLK, OUT), lambda i: (i, 0)),
        out_shape=jax.ShapeDtypeStruct((N, OUT), jnp.float32),
    )(acc, den, bias_out.reshape(1, OUT))
    return out[None, :, :]
